# Initial kernel scaffold; baseline (speedup 1.0000x reference)
import jax
import jax.numpy as jnp
from jax.experimental import pallas as pl


def kernel(edges, poses, nodes):
    raise NotImplementedError("write your pallas kernel here")


# speedup vs baseline: 5.8067x; 5.8067x over previous
import functools

import jax
import jax.numpy as jnp
import numpy as np
from jax import lax
from jax.experimental import pallas as pl
from jax.experimental.pallas import tpu as pltpu
from jax.experimental.pallas import tpu_sc as plsc

NC, NS, L = 2, 16, 16
NW = NC * NS
C = 80

_ATAN_C = (0.9999994367748428, -0.333301063526934, 0.1994850483565558,
           -0.1391577966683494, 0.09656194591100341, -0.05606228170890616,
           0.021945958606742866, -0.004073120782257217)
_PI = float(np.pi)


def _rsqrt(x):
    i = lax.bitcast_convert_type(x, jnp.int32)
    i = jnp.int32(0x5F3759DF) - lax.shift_right_logical(i, 1)
    y = lax.bitcast_convert_type(i, jnp.float32)
    for _ in range(3):
        y = y * (1.5 - 0.5 * x * y * y)
    return y


def _atan2_pos(s, w):
    aw = jnp.abs(w)
    hi = jnp.maximum(s, aw)
    lo = jnp.minimum(s, aw)
    x = lo / jnp.maximum(hi, 1e-30)
    x2 = x * x
    acc = jnp.float32(_ATAN_C[-1])
    for cc in _ATAN_C[-2::-1]:
        acc = acc * x2 + jnp.float32(cc)
    at = acc * x
    a0 = jnp.where(s > aw, jnp.float32(0.5 * _PI) - at, at)
    return jnp.where(w < 0, jnp.float32(_PI) - a0, a0)


def _qmul(a, b):
    ax, ay, az, aw = a
    bx, by, bz, bw = b
    return (aw * bx + ax * bw + ay * bz - az * by,
            aw * by - ax * bz + ay * bw + az * bx,
            aw * bz + ax * by - ay * bx + az * bw,
            aw * bw - ax * bx - ay * by - az * bz)


def _cross(a, b):
    ax, ay, az = a
    bx, by, bz = b
    return (ay * bz - az * by, az * bx - ax * bz, ax * by - ay * bx)


def _qrot(q, v):
    qx, qy, qz, qw = q
    t = _cross((qx, qy, qz), v)
    tx, ty, tz = 2 * t[0], 2 * t[1], 2 * t[2]
    cx, cy, cz = _cross((qx, qy, qz), (tx, ty, tz))
    return (v[0] + qw * tx + cx, v[1] + qw * ty + cy, v[2] + qw * tz + cz)


def _edge_math(p, a, b):
    pt, pq = p[:3], p[3:]
    at_, aq = a[:3], a[3:]
    bt, bq = b[:3], b[3:]
    pqi = (-pq[0], -pq[1], -pq[2], pq[3])
    r = _qrot(pqi, pt)
    pti = (-r[0], -r[1], -r[2])
    aqi = (-aq[0], -aq[1], -aq[2], aq[3])
    r = _qrot(aqi, at_)
    ati = (-r[0], -r[1], -r[2])
    r = _qrot(pqi, ati)
    t1 = (pti[0] + r[0], pti[1] + r[1], pti[2] + r[2])
    q1 = _qmul(pqi, aqi)
    r = _qrot(q1, bt)
    te = (t1[0] + r[0], t1[1] + r[1], t1[2] + r[2])
    ux, uy, uz, w = _qmul(q1, bq)

    s2 = ux * ux + uy * uy + uz * uz
    rs = _rsqrt(jnp.maximum(s2, jnp.float32(1e-30)))
    s = s2 * rs
    theta_q = 2.0 * _atan2_pos(s, w)
    small = s < 1e-6
    s_safe = jnp.where(small, jnp.float32(1.0), s)
    w_safe = jnp.where(jnp.abs(w) < 1e-6, jnp.float32(1.0), w)
    scale = jnp.where(small,
                      (2.0 / w_safe) * (1.0 - s2 / (3.0 * w_safe * w_safe)),
                      theta_q / s_safe)
    px, py, pz = ux * scale, uy * scale, uz * scale
    theta = jnp.abs(scale) * s
    small2 = theta < 1e-6
    theta_safe = jnp.where(small2, jnp.float32(1.0), theta)
    rq = _rsqrt(jnp.maximum(s2 + w * w, jnp.float32(1e-30)))
    sin_half = jnp.where(small, 0.5 * theta_safe, s * rq)
    cos_half = jnp.where(small, jnp.float32(1.0), w * rq)
    sin_half_safe = jnp.where(jnp.abs(sin_half) < 1e-12,
                              jnp.float32(1.0), sin_half)
    c = jnp.where(
        small2,
        1.0 / 12.0 + theta * theta / 720.0,
        (1.0 - theta_safe * cos_half / (2.0 * sin_half_safe))
        / (theta_safe * theta_safe))
    th2 = px * px + py * py + pz * pz
    pdt = px * te[0] + py * te[1] + pz * te[2]
    cx, cy, cz = _cross((px, py, pz), te)
    k1 = 1.0 - c * th2
    taux = k1 * te[0] - 0.5 * cx + c * pdt * px
    tauy = k1 * te[1] - 0.5 * cy + c * pdt * py
    tauz = k1 * te[2] - 0.5 * cz + c * pdt * pz
    return (taux, tauy, tauz, px, py, pz)


def _pg_body(n_edges, edges_hbm, poses_hbm, nodes_hbm, out_hbm,
             e_v, i1_v, i2_v, p_v, n1_v, n2_v, o_v, se, sn1, sn2, sp, so):
    wid = lax.axis_index("s") * NC + lax.axis_index("c")
    per_w = n_edges // NW
    n_chunks = per_w // C
    start = wid * per_w
    col = [jnp.full((L,), j, jnp.int32) for j in range(7)]

    def stage_edges(g, b):
        pltpu.async_copy(edges_hbm.at[:, pl.ds(start + g * C, C)],
                         e_v[b], se[b])

    def wait_edges(g, b):
        pltpu.make_async_copy(edges_hbm.at[:, pl.ds(start + g * C, C)],
                              e_v[b], se[b]).wait()

    def build_and_fire(g, b):
        for k in range(C // L):
            sl = pl.ds(k * L, L)
            i1_v[b][sl] = e_v[b][0, sl]
            i2_v[b][sl] = e_v[b][1, sl]
        pltpu.async_copy(nodes_hbm.at[i1_v[b]], n1_v[b], sn1[b])
        pltpu.async_copy(nodes_hbm.at[i2_v[b]], n2_v[b], sn2[b])
        pltpu.async_copy(poses_hbm.at[:, pl.ds(start + g * C, C)],
                         p_v[b], sp[b])

    def wait_gathers(g, b):
        pltpu.make_async_copy(nodes_hbm.at[i1_v[b]], n1_v[b], sn1[b]).wait()
        pltpu.make_async_copy(nodes_hbm.at[i2_v[b]], n2_v[b], sn2[b]).wait()
        pltpu.make_async_copy(poses_hbm.at[:, pl.ds(start + g * C, C)],
                              p_v[b], sp[b]).wait()

    def wait_out(g, b):
        pltpu.make_async_copy(o_v[b],
                              out_hbm.at[:, pl.ds(start + g * C, C)],
                              so[b]).wait()

    def compute(g, b):
        for k in range(C // L):
            rows = lax.iota(jnp.int32, L) + (k * L)
            sl = pl.ds(k * L, L)
            p = tuple(p_v[b][j, sl] for j in range(7))
            a = tuple(plsc.load_gather(n1_v[b], [rows, col[j]])
                      for j in range(7))
            bb = tuple(plsc.load_gather(n2_v[b], [rows, col[j]])
                       for j in range(7))
            o = _edge_math(p, a, bb)
            for j in range(6):
                o_v[b][j, sl] = o[j]
        pltpu.async_copy(o_v[b], out_hbm.at[:, pl.ds(start + g * C, C)],
                         so[b])

    stage_edges(0, 0)
    wait_edges(0, 0)
    build_and_fire(0, 0)
    stage_edges(1, 1)

    n_pairs = (n_chunks - 1) // 2

    def pair(i, carry):
        a = 2 * i
        wait_edges(a + 1, 1)
        build_and_fire(a + 1, 1)
        stage_edges(a + 2, 0)

        @pl.when(a >= 2)
        def _():
            wait_out(a - 2, 0)

        wait_gathers(a, 0)
        compute(a, 0)
        wait_edges(a + 2, 0)
        build_and_fire(a + 2, 0)

        @pl.when(a + 3 < n_chunks)
        def _():
            stage_edges(a + 3, 1)

        @pl.when(a >= 1)
        def _():
            wait_out(a - 1, 1)

        wait_gathers(a + 1, 1)
        compute(a + 1, 1)
        return carry

    lax.fori_loop(0, n_pairs, pair, 0)

    gl = n_chunks - 1
    wait_out(gl - 2, 0)
    wait_gathers(gl, 0)
    compute(gl, 0)
    wait_out(gl, 0)
    wait_out(gl - 1, 1)


def kernel(edges, poses, nodes):
    n_edges = edges.shape[0]
    mesh = plsc.VectorSubcoreMesh(core_axis_name="c", subcore_axis_name="s")
    run = pl.kernel(
        functools.partial(_pg_body, n_edges),
        out_type=jax.ShapeDtypeStruct((6, n_edges), jnp.float32),
        mesh=mesh,
        compiler_params=pltpu.CompilerParams(
            needs_layout_passes=False, use_tc_tiling_on_sc=False),
        scratch_types=[
            [pltpu.VMEM((2, C), jnp.int32)] * 2,
            [pltpu.VMEM((C,), jnp.int32)] * 2,
            [pltpu.VMEM((C,), jnp.int32)] * 2,
            [pltpu.VMEM((7, C), jnp.float32)] * 2,
            [pltpu.VMEM((C, 8), jnp.float32)] * 2,
            [pltpu.VMEM((C, 8), jnp.float32)] * 2,
            [pltpu.VMEM((6, C), jnp.float32)] * 2,
            [pltpu.SemaphoreType.DMA] * 2,
            [pltpu.SemaphoreType.DMA] * 2,
            [pltpu.SemaphoreType.DMA] * 2,
            [pltpu.SemaphoreType.DMA] * 2,
            [pltpu.SemaphoreType.DMA] * 2,
        ],
    )
    nodes_p = jnp.pad(nodes, ((0, 0), (0, 1)))
    out_t = run(edges.T, poses.T, nodes_p)
    return out_t.T


# speedup vs baseline: 6.8383x; 1.1777x over previous
import functools

import jax
import jax.numpy as jnp
import numpy as np
from jax import lax
from jax.experimental import pallas as pl
from jax.experimental.pallas import tpu as pltpu
from jax.experimental.pallas import tpu_sc as plsc

NC, NS, L = 2, 16, 16
NW = NC * NS
C = 80

_ATAN_C = (0.9999994367748428, -0.333301063526934, 0.1994850483565558,
           -0.1391577966683494, 0.09656194591100341, -0.05606228170890616,
           0.021945958606742866, -0.004073120782257217)
_PI = float(np.pi)


def _rsqrt(x):
    i = lax.bitcast_convert_type(x, jnp.int32)
    i = jnp.int32(0x5F3759DF) - lax.shift_right_logical(i, 1)
    y = lax.bitcast_convert_type(i, jnp.float32)
    for _ in range(3):
        y = y * (1.5 - 0.5 * x * y * y)
    return y


def _atan2_pos(s, w):
    aw = jnp.abs(w)
    hi = jnp.maximum(s, aw)
    lo = jnp.minimum(s, aw)
    x = lo / jnp.maximum(hi, 1e-30)
    x2 = x * x
    acc = jnp.float32(_ATAN_C[-1])
    for cc in _ATAN_C[-2::-1]:
        acc = acc * x2 + jnp.float32(cc)
    at = acc * x
    a0 = jnp.where(s > aw, jnp.float32(0.5 * _PI) - at, at)
    return jnp.where(w < 0, jnp.float32(_PI) - a0, a0)


def _qmul(a, b):
    ax, ay, az, aw = a
    bx, by, bz, bw = b
    return (aw * bx + ax * bw + ay * bz - az * by,
            aw * by - ax * bz + ay * bw + az * bx,
            aw * bz + ax * by - ay * bx + az * bw,
            aw * bw - ax * bx - ay * by - az * bz)


def _cross(a, b):
    ax, ay, az = a
    bx, by, bz = b
    return (ay * bz - az * by, az * bx - ax * bz, ax * by - ay * bx)


def _qrot(q, v):
    qx, qy, qz, qw = q
    t = _cross((qx, qy, qz), v)
    tx, ty, tz = 2 * t[0], 2 * t[1], 2 * t[2]
    cx, cy, cz = _cross((qx, qy, qz), (tx, ty, tz))
    return (v[0] + qw * tx + cx, v[1] + qw * ty + cy, v[2] + qw * tz + cz)


def _edge_math(p, a, b):
    pt, pq = p[:3], p[3:]
    at_, aq = a[:3], a[3:]
    bt, bq = b[:3], b[3:]
    pqi = (-pq[0], -pq[1], -pq[2], pq[3])
    r = _qrot(pqi, pt)
    pti = (-r[0], -r[1], -r[2])
    aqi = (-aq[0], -aq[1], -aq[2], aq[3])
    r = _qrot(aqi, at_)
    ati = (-r[0], -r[1], -r[2])
    r = _qrot(pqi, ati)
    t1 = (pti[0] + r[0], pti[1] + r[1], pti[2] + r[2])
    q1 = _qmul(pqi, aqi)
    r = _qrot(q1, bt)
    te = (t1[0] + r[0], t1[1] + r[1], t1[2] + r[2])
    ux, uy, uz, w = _qmul(q1, bq)

    s2 = ux * ux + uy * uy + uz * uz
    rs = _rsqrt(jnp.maximum(s2, jnp.float32(1e-30)))
    s = s2 * rs
    theta_q = 2.0 * _atan2_pos(s, w)
    small = s < 1e-6
    s_safe = jnp.where(small, jnp.float32(1.0), s)
    w_safe = jnp.where(jnp.abs(w) < 1e-6, jnp.float32(1.0), w)
    scale = jnp.where(small,
                      (2.0 / w_safe) * (1.0 - s2 / (3.0 * w_safe * w_safe)),
                      theta_q / s_safe)
    px, py, pz = ux * scale, uy * scale, uz * scale
    theta = jnp.abs(scale) * s
    small2 = theta < 1e-6
    theta_safe = jnp.where(small2, jnp.float32(1.0), theta)
    rq = _rsqrt(jnp.maximum(s2 + w * w, jnp.float32(1e-30)))
    sin_half = jnp.where(small, 0.5 * theta_safe, s * rq)
    cos_half = jnp.where(small, jnp.float32(1.0), w * rq)
    sin_half_safe = jnp.where(jnp.abs(sin_half) < 1e-12,
                              jnp.float32(1.0), sin_half)
    c = jnp.where(
        small2,
        1.0 / 12.0 + theta * theta / 720.0,
        (1.0 - theta_safe * cos_half / (2.0 * sin_half_safe))
        / (theta_safe * theta_safe))
    th2 = px * px + py * py + pz * pz
    pdt = px * te[0] + py * te[1] + pz * te[2]
    cx, cy, cz = _cross((px, py, pz), te)
    k1 = 1.0 - c * th2
    taux = k1 * te[0] - 0.5 * cx + c * pdt * px
    tauy = k1 * te[1] - 0.5 * cy + c * pdt * py
    tauz = k1 * te[2] - 0.5 * cz + c * pdt * pz
    return (taux, tauy, tauz, px, py, pz)


def _pg_body(e0, n_part, edges_hbm, poses_hbm, nodes_hbm, out_hbm,
             e_v, i1_v, i2_v, p_v, n1_v, n2_v, o_v, se, sn1, sn2, sp, so):
    wid = lax.axis_index("s") * NC + lax.axis_index("c")
    per_w = n_part // NW
    n_chunks = per_w // C
    ostart = wid * per_w
    start = e0 + ostart
    col = [jnp.full((L,), j, jnp.int32) for j in range(7)]

    def stage_edges(g, b):
        pltpu.async_copy(edges_hbm.at[:, pl.ds(start + g * C, C)],
                         e_v[b], se[b])

    def wait_edges(g, b):
        pltpu.make_async_copy(edges_hbm.at[:, pl.ds(start + g * C, C)],
                              e_v[b], se[b]).wait()

    def build_and_fire(g, b):
        for k in range(C // L):
            sl = pl.ds(k * L, L)
            i1_v[b][sl] = e_v[b][0, sl]
            i2_v[b][sl] = e_v[b][1, sl]
        pltpu.async_copy(nodes_hbm.at[i1_v[b]], n1_v[b], sn1[b])
        pltpu.async_copy(nodes_hbm.at[i2_v[b]], n2_v[b], sn2[b])
        pltpu.async_copy(poses_hbm.at[:, pl.ds(start + g * C, C)],
                         p_v[b], sp[b])

    def wait_gathers(g, b):
        pltpu.make_async_copy(nodes_hbm.at[i1_v[b]], n1_v[b], sn1[b]).wait()
        pltpu.make_async_copy(nodes_hbm.at[i2_v[b]], n2_v[b], sn2[b]).wait()
        pltpu.make_async_copy(poses_hbm.at[:, pl.ds(start + g * C, C)],
                              p_v[b], sp[b]).wait()

    def wait_out(g, b):
        pltpu.make_async_copy(o_v[b],
                              out_hbm.at[:, pl.ds(ostart + g * C, C)],
                              so[b]).wait()

    def compute(g, b):
        for k in range(C // L):
            rows = lax.iota(jnp.int32, L) + (k * L)
            sl = pl.ds(k * L, L)
            p = tuple(p_v[b][j, sl] for j in range(7))
            a = tuple(plsc.load_gather(n1_v[b], [rows, col[j]])
                      for j in range(7))
            bb = tuple(plsc.load_gather(n2_v[b], [rows, col[j]])
                       for j in range(7))
            o = _edge_math(p, a, bb)
            for j in range(6):
                o_v[b][j, sl] = o[j]
        pltpu.async_copy(o_v[b], out_hbm.at[:, pl.ds(ostart + g * C, C)],
                         so[b])

    stage_edges(0, 0)
    wait_edges(0, 0)
    build_and_fire(0, 0)
    stage_edges(1, 1)

    n_pairs = (n_chunks - 1) // 2

    def pair(i, carry):
        a = 2 * i
        wait_edges(a + 1, 1)
        build_and_fire(a + 1, 1)
        stage_edges(a + 2, 0)

        @pl.when(a >= 2)
        def _():
            wait_out(a - 2, 0)

        wait_gathers(a, 0)
        compute(a, 0)
        wait_edges(a + 2, 0)
        build_and_fire(a + 2, 0)

        @pl.when(a + 3 < n_chunks)
        def _():
            stage_edges(a + 3, 1)

        @pl.when(a >= 1)
        def _():
            wait_out(a - 1, 1)

        wait_gathers(a + 1, 1)
        compute(a + 1, 1)
        return carry

    lax.fori_loop(0, n_pairs, pair, 0)

    if n_chunks % 2 == 1:
        gl = n_chunks - 1
        wait_out(gl - 2, 0)
        wait_gathers(gl, 0)
        compute(gl, 0)
        wait_out(gl, 0)
        wait_out(gl - 1, 1)
    else:
        m = n_chunks - 2
        wait_edges(m + 1, 1)
        build_and_fire(m + 1, 1)
        wait_out(m - 2, 0)
        wait_gathers(m, 0)
        compute(m, 0)
        wait_out(m - 1, 1)
        wait_gathers(m + 1, 1)
        compute(m + 1, 1)
        wait_out(m, 0)
        wait_out(m + 1, 1)


def _make_run(e0, n_part):
    mesh = plsc.VectorSubcoreMesh(core_axis_name="c", subcore_axis_name="s")
    return pl.kernel(
        functools.partial(_pg_body, e0, n_part),
        out_type=jax.ShapeDtypeStruct((6, n_part), jnp.float32),
        mesh=mesh,
        compiler_params=pltpu.CompilerParams(
            needs_layout_passes=False, use_tc_tiling_on_sc=False),
        scratch_types=[
            [pltpu.VMEM((2, C), jnp.int32)] * 2,
            [pltpu.VMEM((C,), jnp.int32)] * 2,
            [pltpu.VMEM((C,), jnp.int32)] * 2,
            [pltpu.VMEM((7, C), jnp.float32)] * 2,
            [pltpu.VMEM((C, 8), jnp.float32)] * 2,
            [pltpu.VMEM((C, 8), jnp.float32)] * 2,
            [pltpu.VMEM((6, C), jnp.float32)] * 2,
            [pltpu.SemaphoreType.DMA] * 2,
            [pltpu.SemaphoreType.DMA] * 2,
            [pltpu.SemaphoreType.DMA] * 2,
            [pltpu.SemaphoreType.DMA] * 2,
            [pltpu.SemaphoreType.DMA] * 2,
        ],
    )


def kernel(edges, poses, nodes):
    n_edges = edges.shape[0]
    nodes_p = jnp.pad(nodes, ((0, 0), (0, 1)))
    edges_t = edges.T
    poses_t = poses.T
    na = (n_edges * 13) // 25 // (NW * C) * (NW * C)
    nb = n_edges - na
    if nb <= 0 or na <= 0 or nb % (NW * C) != 0:
        na, nb = n_edges, 0
    out_a = _make_run(0, na)(edges_t, poses_t, nodes_p)
    if nb:
        out_b = _make_run(na, nb)(edges_t, poses_t, nodes_p)
        return jnp.concatenate([out_a, out_b], axis=1).T
    return out_a.T
